# baseline (speedup 1.0000x reference)
import functools

import jax
import jax.numpy as jnp
from jax import lax
from jax.experimental import pallas as pl
from jax.experimental.pallas import tpu as pltpu
from jax.experimental.pallas import tpu_sc as plsc

N = 10000
E = 160000
NROWS_PAD = 10240
CHUNK = 32
PAIR = 2 * CHUNK
E_PAD = 160064
TILE_EDGE_BUF = 5056


def _mm_act_kern(slope, x_ref, w_ref, b_ref, o_ref):
    o = jnp.dot(x_ref[...], w_ref[...], preferred_element_type=jnp.float32)
    o = o + b_ref[...]
    if slope is not None:
        o = jnp.where(o > 0, o, slope * o)
    o_ref[...] = o


def _pallas_matmul(x, w, b, slope=None, block_rows=1000):
    n, k = x.shape
    _, m = w.shape
    return pl.pallas_call(
        functools.partial(_mm_act_kern, slope),
        grid=(n // block_rows,),
        in_specs=[
            pl.BlockSpec((block_rows, k), lambda i: (i, 0)),
            pl.BlockSpec((k, m), lambda i: (0, 0)),
            pl.BlockSpec((m,), lambda i: (0,)),
        ],
        out_specs=pl.BlockSpec((block_rows, m), lambda i: (i, 0)),
        out_shape=jax.ShapeDtypeStruct((n, m), jnp.float32),
    )(x, w, b)


def _gat_mm_kern(npk, x_ref, w_ref, b_ref, as_ref, ad_ref, z_ref, ss_ref, sd_ref):
    po = pl.program_id(1)
    pk = pl.program_id(2)
    part = jnp.dot(x_ref[0], w_ref[0, 0], preferred_element_type=jnp.float32)

    @pl.when(pk == 0)
    def _():
        z_ref[0] = part + b_ref[0]

    @pl.when(pk > 0)
    def _():
        z_ref[0] = z_ref[0] + part

    @pl.when(pk == npk - 1)
    def _():
        zfull = z_ref[0]
        ssp = jnp.dot(zfull, as_ref[0, 0], preferred_element_type=jnp.float32)
        sdp = jnp.dot(zfull, ad_ref[0, 0], preferred_element_type=jnp.float32)

        @pl.when(po == 0)
        def _():
            ss_ref[...] = ssp[:, None]
            sd_ref[...] = sdp[:, None]

        @pl.when(po > 0)
        def _():
            ss_ref[...] = ss_ref[...] + ssp[:, None]
            sd_ref[...] = sd_ref[...] + sdp[:, None]


def _gat_matmul(xp, w4, b2, a_s2, a_d2, wp, block_rows=1000):
    p, n, _ = xp.shape
    grid = (n // block_rows, p, p)
    return pl.pallas_call(
        functools.partial(_gat_mm_kern, p),
        grid=grid,
        in_specs=[
            pl.BlockSpec((1, block_rows, wp), lambda i, po, pk: (pk, i, 0)),
            pl.BlockSpec((1, 1, wp, wp), lambda i, po, pk: (pk, po, 0, 0)),
            pl.BlockSpec((1, 1, wp), lambda i, po, pk: (po, 0, 0)),
            pl.BlockSpec((1, 1, wp), lambda i, po, pk: (po, 0, 0)),
            pl.BlockSpec((1, 1, wp), lambda i, po, pk: (po, 0, 0)),
        ],
        out_specs=[
            pl.BlockSpec((1, block_rows, wp), lambda i, po, pk: (po, i, 0)),
            pl.BlockSpec((block_rows, 1), lambda i, po, pk: (i, 0)),
            pl.BlockSpec((block_rows, 1), lambda i, po, pk: (i, 0)),
        ],
        out_shape=[
            jax.ShapeDtypeStruct((p, n, wp), jnp.float32),
            jax.ShapeDtypeStruct((n, 1), jnp.float32),
            jax.ShapeDtypeStruct((n, 1), jnp.float32),
        ],
    )(xp, w4, b2, a_s2, a_d2)


def _resid_kern(wp, p, dp, h_ref, agg_ref, sn_ref, o_ref):
    a = agg_ref[...]
    s = a[0] + a[1]
    den = s[p][:, 0]
    inv = 1.0 / (den + 1e-9)
    parts = [s[q] for q in range(p)]
    aggc = jnp.concatenate(parts, axis=1)
    o = (h_ref[...] + aggc * inv[:, None]) * sn_ref[...]
    o_ref[...] = jnp.maximum(o, 0.0)


def _gat_resid(h_pad, agg, snorm2d, wp, block_rows=400):
    n, dp = h_pad.shape
    p = dp // wp
    return pl.pallas_call(
        functools.partial(_resid_kern, wp, p, dp),
        grid=(n // block_rows,),
        in_specs=[
            pl.BlockSpec((block_rows, dp), lambda i: (i, 0)),
            pl.BlockSpec((2, p + 1, block_rows, wp), lambda i: (0, 0, i, 0)),
            pl.BlockSpec((block_rows, 1), lambda i: (i, 0)),
        ],
        out_specs=pl.BlockSpec((block_rows, dp), lambda i: (i, 0)),
        out_shape=jax.ShapeDtypeStruct((n, dp), jnp.float32),
    )(h_pad, agg, snorm2d)


@functools.lru_cache(maxsize=None)
def _make_sc_edge_kernel(p):
    wp = 128
    rows_per_tile = NROWS_PAD // 16
    mesh = plsc.VectorSubcoreMesh(core_axis_name="c", subcore_axis_name="s")

    @functools.partial(
        pl.kernel,
        mesh=mesh,
        compiler_params=pltpu.CompilerParams(needs_layout_passes=False),
        out_type=jax.ShapeDtypeStruct((2, p + 1, NROWS_PAD, wp), jnp.float32),
        scratch_types=[
            pltpu.VMEM((TILE_EDGE_BUF,), jnp.int32),
            pltpu.VMEM((TILE_EDGE_BUF,), jnp.int32),
            pltpu.VMEM((TILE_EDGE_BUF,), jnp.float32),
            pltpu.VMEM((TILE_EDGE_BUF,), jnp.float32),
            pltpu.VMEM((16,), jnp.float32),
            pltpu.VMEM((PAIR,), jnp.float32),
            pltpu.VMEM((PAIR,), jnp.float32),
            pltpu.VMEM((CHUNK,), jnp.int32),
            pltpu.VMEM((CHUNK,), jnp.int32),
            pltpu.VMEM((CHUNK, wp), jnp.float32),
            pltpu.VMEM((CHUNK, wp), jnp.float32),
            pltpu.VMEM((CHUNK, wp), jnp.float32),
            pltpu.VMEM((CHUNK, wp), jnp.float32),
            pltpu.VMEM_SHARED((NROWS_PAD, wp), jnp.float32),
            pltpu.SemaphoreType.DMA,
            pltpu.SemaphoreType.DMA,
            pltpu.SemaphoreType.DMA,
            pltpu.SemaphoreType.DMA,
        ],
    )
    def sc_edge(z_hbm, ss_hbm, sd_hbm, src_hbm, dst_hbm, ew_hbm, c_hbm,
                zeros_hbm, agg_hbm,
                src_v, dst_v, ew_v, ex_v, c_v, ssg_v, sdg_v,
                sidx_a, sidx_b,
                rows_a, rows_b, scl_a, scl_b,
                acc_sh, gsem_a, gsem_b, ssem_a, ssem_b):
        cid = lax.axis_index("c")
        sid = lax.axis_index("s")
        wid = sid * 2 + cid

        npairs = jnp.where(wid < 4, 79, 78)
        base_pair = jnp.where(wid < 4, 79 * wid, 316 + 78 * (wid - 4))
        base_e = base_pair * PAIR

        pltpu.sync_copy(src_hbm.at[pl.ds(base_e, TILE_EDGE_BUF)], src_v)
        pltpu.sync_copy(dst_hbm.at[pl.ds(base_e, TILE_EDGE_BUF)], dst_v)
        pltpu.sync_copy(ew_hbm.at[pl.ds(base_e, TILE_EDGE_BUF)], ew_v)
        pltpu.sync_copy(c_hbm, c_v)
        cval = c_v[...]

        def ex_body(g, carry):
            o = g * PAIR
            ga = pltpu.async_copy(ss_hbm.at[src_v.at[pl.ds(o, PAIR)]], ssg_v, gsem_a)
            gb = pltpu.async_copy(sd_hbm.at[dst_v.at[pl.ds(o, PAIR)]], sdg_v, gsem_b)
            ga.wait()
            gb.wait()
            for k in range(PAIR // 16):
                lg = (ssg_v[pl.ds(k * 16, 16)]
                      + sdg_v[pl.ds(k * 16, 16)]
                      + cval * ew_v[pl.ds(o + k * 16, 16)])
                lg = jnp.maximum(lg, 0.2 * lg)
                ex_v[pl.ds(o + k * 16, 16)] = jnp.exp(lg)
            return carry

        lax.fori_loop(0, npairs, ex_body, 0)

        zero16 = jnp.zeros((16,), jnp.int32)
        UNROLL = 4

        def scale_chunk(rows_v, scl_v, ebase):
            def r_body(u, carry):
                r0 = u * UNROLL
                for r in range(UNROLL):
                    spl = plsc.load_gather(ex_v, [zero16 + (ebase + r0 + r)])
                    for k in range(wp // 16):
                        scl_v[r0 + r, pl.ds(k * 16, 16)] = rows_v[r0 + r, pl.ds(k * 16, 16)] * spl
                return carry
            lax.fori_loop(0, CHUNK // UNROLL, r_body, 0)

        def splat_chunk(scl_v, ebase):
            def r_body(u, carry):
                r0 = u * UNROLL
                for r in range(UNROLL):
                    spl = plsc.load_gather(ex_v, [zero16 + (ebase + r0 + r)])
                    for k in range(wp // 16):
                        scl_v[r0 + r, pl.ds(k * 16, 16)] = spl
                return carry
            lax.fori_loop(0, CHUNK // UNROLL, r_body, 0)

        def stage_idx(dst_buf, src_ref, off):
            for k in range(CHUNK // 16):
                dst_buf[pl.ds(k * 16, 16)] = src_ref[pl.ds(off + k * 16, 16)]

        for q in range(p + 1):
            pltpu.sync_copy(zeros_hbm, acc_sh.at[pl.ds(sid * rows_per_tile, rows_per_tile)])
            plsc.subcore_barrier()

            if q < p:
                zq = z_hbm.at[q]

                def pair_body(jp, carry):
                    ea = jp * PAIR
                    eb = ea + CHUNK
                    ga = pltpu.async_copy(zq.at[src_v.at[pl.ds(ea, CHUNK)]], rows_a, gsem_a)
                    gb = pltpu.async_copy(zq.at[src_v.at[pl.ds(eb, CHUNK)]], rows_b, gsem_b)
                    ga.wait()
                    scale_chunk(rows_a, scl_a, ea)
                    stage_idx(sidx_a, dst_v, ea)
                    sa = pltpu.async_copy(scl_a, acc_sh.at[sidx_a], ssem_a, add=True)
                    gb.wait()
                    scale_chunk(rows_b, scl_b, eb)
                    stage_idx(sidx_b, dst_v, eb)
                    sb = pltpu.async_copy(scl_b, acc_sh.at[sidx_b], ssem_b, add=True)
                    sa.wait()
                    sb.wait()
                    return carry
            else:
                def pair_body(jp, carry):
                    ea = jp * PAIR
                    eb = ea + CHUNK
                    splat_chunk(scl_a, ea)
                    stage_idx(sidx_a, dst_v, ea)
                    sa = pltpu.async_copy(scl_a, acc_sh.at[sidx_a], ssem_a, add=True)
                    splat_chunk(scl_b, eb)
                    stage_idx(sidx_b, dst_v, eb)
                    sb = pltpu.async_copy(scl_b, acc_sh.at[sidx_b], ssem_b, add=True)
                    sa.wait()
                    sb.wait()
                    return carry

            lax.fori_loop(0, npairs, pair_body, 0)
            plsc.subcore_barrier()
            pltpu.sync_copy(
                acc_sh.at[pl.ds(sid * rows_per_tile, rows_per_tile)],
                agg_hbm.at[cid, q, pl.ds(sid * rows_per_tile, rows_per_tile)],
            )
            if q + 1 < p + 1:
                plsc.subcore_barrier()

    return sc_edge


def _prep_gat_params(pp, pre, d, dp, wp, att_ew):
    p = dp // wp
    wm = pp[pre + "_W"]
    b = pp[pre + "_b"]
    a = pp[pre + "_a"][:, 0]
    w_ext = jnp.pad(wm, ((0, dp - d), (0, dp - d)))
    w4 = w_ext.reshape(p, wp, p, wp).transpose(0, 2, 1, 3)
    b2 = jnp.pad(b, (0, dp - d)).reshape(p, 1, wp)
    a_s2 = jnp.pad(a[:d], (0, dp - d)).reshape(p, 1, wp)
    a_d2 = jnp.pad(a[d:2 * d], (0, dp - d)).reshape(p, 1, wp)
    if att_ew:
        cvec = jnp.full((16,), jnp.sum(a[2 * d:]), jnp.float32)
    else:
        cvec = jnp.zeros((16,), jnp.float32)
    return w4, b2, a_s2, a_d2, cvec


def _gat_layer(h_pad, edges, lp, snorm2d, wp):
    w4, b2, a_s2, a_d2, cvec = lp
    src_p, dst_p, ew_p, zeros_hbm = edges
    n, dp = h_pad.shape
    p = dp // wp
    xp = h_pad.reshape(n, p, wp).transpose(1, 0, 2)
    zpad, ss, sd = _gat_matmul(xp, w4, b2, a_s2, a_d2, wp)
    sc_k = _make_sc_edge_kernel(p)
    agg = sc_k(zpad, ss.reshape(n), sd.reshape(n), src_p, dst_p, ew_p, cvec, zeros_hbm)
    return _gat_resid(h_pad, agg, snorm2d, wp)


def kernel(feats, e_w, snorm_n, gt, maps_emb, params, edge_index):
    p = params
    src = edge_index[0]
    dst = edge_index[1]
    src_p = jnp.pad(src, (0, E_PAD - E))
    dst_p = jnp.pad(dst, (0, E_PAD - E))
    ew_p = jnp.pad(e_w[:, 0], (0, E_PAD - E))
    snorm2d = snorm_n

    h_emb = _pallas_matmul(feats, p["emb_W"], p["emb_b"])

    def run_stack(h0, pre, d, dp, wp, att_ew):
        h_pad = jnp.pad(h0, ((0, 0), (0, dp - d)))
        zeros_hbm = jnp.zeros((NROWS_PAD // 16, wp), jnp.float32)
        edges = (src_p, dst_p, ew_p, zeros_hbm)
        lp1 = _prep_gat_params(p, pre + "1", d, dp, wp, att_ew)
        lp2 = _prep_gat_params(p, pre + "2", d, dp, wp, att_ew)
        h_pad = _gat_layer(h_pad, edges, lp1, snorm2d, wp)
        h_pad = _gat_layer(h_pad, edges, lp2, snorm2d, wp)
        return h_pad[:, :d]

    h0 = jnp.concatenate([maps_emb, h_emb, gt], axis=-1)
    h = run_stack(h0, "enc", 572, 640, 128, True)
    he = jnp.concatenate([h, gt], axis=-1)
    he = _pallas_matmul(he, p["encl_W"], p["encl_b"], slope=0.01)
    mu = _pallas_matmul(he, p["encmu_W"], p["encmu_b"])
    log_var = _pallas_matmul(he, p["enclv_W"], p["enclv_b"])
    hp0 = jnp.concatenate([maps_emb, h_emb], axis=-1)
    hp = run_stack(hp0, "pri", 512, 512, 128, True)
    hp2 = _pallas_matmul(hp, p["pril_W"], p["pril_b"], slope=0.01)
    mu_p = _pallas_matmul(hp2, p["primu_W"], p["primu_b"])
    log_var_p = _pallas_matmul(hp2, p["prilv_W"], p["prilv_b"])
    eps = jax.random.normal(jax.random.key(42), mu.shape, dtype=jnp.float32)
    z = mu + jnp.exp(0.5 * log_var) * eps
    hd0 = jnp.concatenate([h_emb, z], axis=-1)
    hd = run_stack(hd0, "dec", 384, 384, 128, False)
    recon = _pallas_matmul(jnp.concatenate([hd, z], axis=-1), p["out_W"], p["out_b"])
    return (recon, mu, log_var, mu_p, log_var_p)


# speedup vs baseline: 1.7140x; 1.7140x over previous
import functools

import jax
import jax.numpy as jnp
from jax import lax
from jax.experimental import pallas as pl
from jax.experimental.pallas import tpu as pltpu
from jax.experimental.pallas import tpu_sc as plsc

N = 10000
E = 160000
NROWS_PAD = 10240
CHUNK = 32
PAIR = 2 * CHUNK
E_PAD = 160064
TILE_EDGE_BUF = 5056


def _mm_act_kern(slope, x_ref, w_ref, b_ref, o_ref):
    o = jnp.dot(x_ref[...], w_ref[...], preferred_element_type=jnp.float32)
    o = o + b_ref[...]
    if slope is not None:
        o = jnp.where(o > 0, o, slope * o)
    o_ref[...] = o


def _pallas_matmul(x, w, b, slope=None, block_rows=1000):
    n, k = x.shape
    _, m = w.shape
    return pl.pallas_call(
        functools.partial(_mm_act_kern, slope),
        grid=(n // block_rows,),
        in_specs=[
            pl.BlockSpec((block_rows, k), lambda i: (i, 0)),
            pl.BlockSpec((k, m), lambda i: (0, 0)),
            pl.BlockSpec((m,), lambda i: (0,)),
        ],
        out_specs=pl.BlockSpec((block_rows, m), lambda i: (i, 0)),
        out_shape=jax.ShapeDtypeStruct((n, m), jnp.float32),
    )(x, w, b)


def _gat_mm_kern(npk, x_ref, w_ref, b_ref, as_ref, ad_ref, z_ref, ss_ref, sd_ref):
    po = pl.program_id(1)
    pk = pl.program_id(2)
    part = jnp.dot(x_ref[0], w_ref[0, 0], preferred_element_type=jnp.float32)

    @pl.when(pk == 0)
    def _():
        z_ref[0] = part + b_ref[0]

    @pl.when(pk > 0)
    def _():
        z_ref[0] = z_ref[0] + part

    @pl.when(pk == npk - 1)
    def _():
        zfull = z_ref[0]
        ssp = jnp.dot(zfull, as_ref[0, 0], preferred_element_type=jnp.float32)
        sdp = jnp.dot(zfull, ad_ref[0, 0], preferred_element_type=jnp.float32)

        @pl.when(po == 0)
        def _():
            ss_ref[...] = ssp[:, None]
            sd_ref[...] = sdp[:, None]

        @pl.when(po > 0)
        def _():
            ss_ref[...] = ss_ref[...] + ssp[:, None]
            sd_ref[...] = sd_ref[...] + sdp[:, None]


def _gat_matmul(xp, w4, b2, a_s2, a_d2, wp, block_rows=1000):
    p, n, _ = xp.shape
    grid = (n // block_rows, p, p)
    return pl.pallas_call(
        functools.partial(_gat_mm_kern, p),
        grid=grid,
        in_specs=[
            pl.BlockSpec((1, block_rows, wp), lambda i, po, pk: (pk, i, 0)),
            pl.BlockSpec((1, 1, wp, wp), lambda i, po, pk: (pk, po, 0, 0)),
            pl.BlockSpec((1, 1, wp), lambda i, po, pk: (po, 0, 0)),
            pl.BlockSpec((1, 1, wp), lambda i, po, pk: (po, 0, 0)),
            pl.BlockSpec((1, 1, wp), lambda i, po, pk: (po, 0, 0)),
        ],
        out_specs=[
            pl.BlockSpec((1, block_rows, wp), lambda i, po, pk: (po, i, 0)),
            pl.BlockSpec((block_rows, 1), lambda i, po, pk: (i, 0)),
            pl.BlockSpec((block_rows, 1), lambda i, po, pk: (i, 0)),
        ],
        out_shape=[
            jax.ShapeDtypeStruct((p, n, wp), jnp.float32),
            jax.ShapeDtypeStruct((n, 1), jnp.float32),
            jax.ShapeDtypeStruct((n, 1), jnp.float32),
        ],
    )(xp, w4, b2, a_s2, a_d2)


def _resid_kern(wp, p, dp, h_ref, agg_ref, sn_ref, o_ref):
    a = agg_ref[...]
    s = a[0] + a[1]
    den = s[p][:, 0]
    inv = 1.0 / (den + 1e-9)
    parts = [s[q] for q in range(p)]
    aggc = jnp.concatenate(parts, axis=1)
    o = (h_ref[...] + aggc * inv[:, None]) * sn_ref[...]
    o_ref[...] = jnp.maximum(o, 0.0)


def _gat_resid(h_pad, agg, snorm2d, wp, block_rows=400):
    n, dp = h_pad.shape
    p = dp // wp
    return pl.pallas_call(
        functools.partial(_resid_kern, wp, p, dp),
        grid=(n // block_rows,),
        in_specs=[
            pl.BlockSpec((block_rows, dp), lambda i: (i, 0)),
            pl.BlockSpec((2, p + 1, block_rows, wp), lambda i: (0, 0, i, 0)),
            pl.BlockSpec((block_rows, 1), lambda i: (i, 0)),
        ],
        out_specs=pl.BlockSpec((block_rows, dp), lambda i: (i, 0)),
        out_shape=jax.ShapeDtypeStruct((n, dp), jnp.float32),
    )(h_pad, agg, snorm2d)


@functools.lru_cache(maxsize=None)
def _make_sc_edge_kernel(p):
    wp = 128
    rows_per_tile = NROWS_PAD // 16
    mesh = plsc.VectorSubcoreMesh(core_axis_name="c", subcore_axis_name="s")

    @functools.partial(
        pl.kernel,
        mesh=mesh,
        compiler_params=pltpu.CompilerParams(needs_layout_passes=False),
        out_type=jax.ShapeDtypeStruct((2, p + 1, NROWS_PAD, wp), jnp.float32),
        scratch_types=[
            pltpu.VMEM((TILE_EDGE_BUF,), jnp.int32),
            pltpu.VMEM((TILE_EDGE_BUF,), jnp.int32),
            pltpu.VMEM((TILE_EDGE_BUF,), jnp.float32),
            pltpu.VMEM((TILE_EDGE_BUF,), jnp.float32),
            pltpu.VMEM((16,), jnp.float32),
            pltpu.VMEM((PAIR,), jnp.float32),
            pltpu.VMEM((PAIR,), jnp.float32),
            pltpu.VMEM((CHUNK,), jnp.int32),
            pltpu.VMEM((CHUNK,), jnp.int32),
            pltpu.VMEM((CHUNK, wp), jnp.float32),
            pltpu.VMEM((CHUNK, wp), jnp.float32),
            pltpu.VMEM((CHUNK, wp), jnp.float32),
            pltpu.VMEM((CHUNK, wp), jnp.float32),
            pltpu.VMEM_SHARED((NROWS_PAD, wp), jnp.float32),
            pltpu.SemaphoreType.DMA,
            pltpu.SemaphoreType.DMA,
            pltpu.SemaphoreType.DMA,
            pltpu.SemaphoreType.DMA,
        ],
    )
    def sc_edge(z_hbm, ss_hbm, sd_hbm, src_hbm, dst_hbm, ew_hbm, c_hbm,
                zeros_hbm, agg_hbm,
                src_v, dst_v, ew_v, ex_v, c_v, ssg_v, sdg_v,
                sidx_a, sidx_b,
                rows_a, rows_b, scl_a, scl_b,
                acc_sh, gsem_a, gsem_b, ssem_a, ssem_b):
        cid = lax.axis_index("c")
        sid = lax.axis_index("s")
        wid = sid * 2 + cid

        npairs = jnp.where(wid < 4, 79, 78)
        base_pair = jnp.where(wid < 4, 79 * wid, 316 + 78 * (wid - 4))
        base_e = base_pair * PAIR

        pltpu.sync_copy(src_hbm.at[pl.ds(base_e, TILE_EDGE_BUF)], src_v)
        pltpu.sync_copy(dst_hbm.at[pl.ds(base_e, TILE_EDGE_BUF)], dst_v)
        pltpu.sync_copy(ew_hbm.at[pl.ds(base_e, TILE_EDGE_BUF)], ew_v)
        pltpu.sync_copy(c_hbm, c_v)
        cval = c_v[...]

        def ex_body(g, carry):
            o = g * PAIR
            ga = pltpu.async_copy(ss_hbm.at[src_v.at[pl.ds(o, PAIR)]], ssg_v, gsem_a)
            gb = pltpu.async_copy(sd_hbm.at[dst_v.at[pl.ds(o, PAIR)]], sdg_v, gsem_b)
            ga.wait()
            gb.wait()
            for k in range(PAIR // 16):
                lg = (ssg_v[pl.ds(k * 16, 16)]
                      + sdg_v[pl.ds(k * 16, 16)]
                      + cval * ew_v[pl.ds(o + k * 16, 16)])
                lg = jnp.maximum(lg, 0.2 * lg)
                ex_v[pl.ds(o + k * 16, 16)] = jnp.exp(lg)
            return carry

        lax.fori_loop(0, npairs, ex_body, 0)

        zero16 = jnp.zeros((16,), jnp.int32)

        def scale_chunk(rows_v, scl_v, ebase):
            def r_body(r, carry):
                spl = plsc.load_gather(ex_v, [zero16 + (ebase + r)])
                for k in range(wp // 16):
                    scl_v[r, pl.ds(k * 16, 16)] = rows_v[r, pl.ds(k * 16, 16)] * spl
                return carry
            lax.fori_loop(0, CHUNK, r_body, 0)

        def splat_chunk(scl_v, ebase):
            def r_body(r, carry):
                spl = plsc.load_gather(ex_v, [zero16 + (ebase + r)])
                for k in range(wp // 16):
                    scl_v[r, pl.ds(k * 16, 16)] = spl
                return carry
            lax.fori_loop(0, CHUNK, r_body, 0)

        def stage_idx(dst_buf, src_ref, off):
            for k in range(CHUNK // 16):
                dst_buf[pl.ds(k * 16, 16)] = src_ref[pl.ds(off + k * 16, 16)]

        for q in range(p + 1):
            pltpu.sync_copy(zeros_hbm, acc_sh.at[pl.ds(sid * rows_per_tile, rows_per_tile)])
            plsc.subcore_barrier()

            def wait_scatters(jp):
                @pl.when(jp > 0)
                def _():
                    pltpu.make_async_copy(scl_a, acc_sh.at[sidx_a], ssem_a).wait()
                    pltpu.make_async_copy(scl_b, acc_sh.at[sidx_b], ssem_b).wait()

            if q < p:
                zq = z_hbm.at[q]

                def pair_body(jp, carry):
                    ea = jp * PAIR
                    eb = ea + CHUNK
                    ga = pltpu.async_copy(zq.at[src_v.at[pl.ds(ea, CHUNK)]], rows_a, gsem_a)
                    gb = pltpu.async_copy(zq.at[src_v.at[pl.ds(eb, CHUNK)]], rows_b, gsem_b)
                    wait_scatters(jp)
                    ga.wait()
                    scale_chunk(rows_a, scl_a, ea)
                    stage_idx(sidx_a, dst_v, ea)
                    pltpu.async_copy(scl_a, acc_sh.at[sidx_a], ssem_a, add=True)
                    gb.wait()
                    scale_chunk(rows_b, scl_b, eb)
                    stage_idx(sidx_b, dst_v, eb)
                    pltpu.async_copy(scl_b, acc_sh.at[sidx_b], ssem_b, add=True)
                    return carry
            else:
                def pair_body(jp, carry):
                    ea = jp * PAIR
                    eb = ea + CHUNK
                    wait_scatters(jp)
                    splat_chunk(scl_a, ea)
                    stage_idx(sidx_a, dst_v, ea)
                    pltpu.async_copy(scl_a, acc_sh.at[sidx_a], ssem_a, add=True)
                    splat_chunk(scl_b, eb)
                    stage_idx(sidx_b, dst_v, eb)
                    pltpu.async_copy(scl_b, acc_sh.at[sidx_b], ssem_b, add=True)
                    return carry

            lax.fori_loop(0, npairs, pair_body, 0)
            pltpu.make_async_copy(scl_a, acc_sh.at[sidx_a], ssem_a).wait()
            pltpu.make_async_copy(scl_b, acc_sh.at[sidx_b], ssem_b).wait()
            plsc.subcore_barrier()
            pltpu.sync_copy(
                acc_sh.at[pl.ds(sid * rows_per_tile, rows_per_tile)],
                agg_hbm.at[cid, q, pl.ds(sid * rows_per_tile, rows_per_tile)],
            )
            if q + 1 < p + 1:
                plsc.subcore_barrier()

    return sc_edge


def _prep_gat_params(pp, pre, d, dp, wp, att_ew):
    p = dp // wp
    wm = pp[pre + "_W"]
    b = pp[pre + "_b"]
    a = pp[pre + "_a"][:, 0]
    w_ext = jnp.pad(wm, ((0, dp - d), (0, dp - d)))
    w4 = w_ext.reshape(p, wp, p, wp).transpose(0, 2, 1, 3)
    b2 = jnp.pad(b, (0, dp - d)).reshape(p, 1, wp)
    a_s2 = jnp.pad(a[:d], (0, dp - d)).reshape(p, 1, wp)
    a_d2 = jnp.pad(a[d:2 * d], (0, dp - d)).reshape(p, 1, wp)
    if att_ew:
        cvec = jnp.full((16,), jnp.sum(a[2 * d:]), jnp.float32)
    else:
        cvec = jnp.zeros((16,), jnp.float32)
    return w4, b2, a_s2, a_d2, cvec


def _gat_layer(h_pad, edges, lp, snorm2d, wp):
    w4, b2, a_s2, a_d2, cvec = lp
    src_p, dst_p, ew_p, zeros_hbm = edges
    n, dp = h_pad.shape
    p = dp // wp
    xp = h_pad.reshape(n, p, wp).transpose(1, 0, 2)
    zpad, ss, sd = _gat_matmul(xp, w4, b2, a_s2, a_d2, wp)
    sc_k = _make_sc_edge_kernel(p)
    agg = sc_k(zpad, ss.reshape(n), sd.reshape(n), src_p, dst_p, ew_p, cvec, zeros_hbm)
    return _gat_resid(h_pad, agg, snorm2d, wp)


def kernel(feats, e_w, snorm_n, gt, maps_emb, params, edge_index):
    p = params
    src = edge_index[0]
    dst = edge_index[1]
    src_p = jnp.pad(src, (0, E_PAD - E))
    dst_p = jnp.pad(dst, (0, E_PAD - E))
    ew_p = jnp.pad(e_w[:, 0], (0, E_PAD - E))
    snorm2d = snorm_n

    h_emb = _pallas_matmul(feats, p["emb_W"], p["emb_b"])

    def run_stack(h0, pre, d, dp, wp, att_ew):
        h_pad = jnp.pad(h0, ((0, 0), (0, dp - d)))
        zeros_hbm = jnp.zeros((NROWS_PAD // 16, wp), jnp.float32)
        edges = (src_p, dst_p, ew_p, zeros_hbm)
        lp1 = _prep_gat_params(p, pre + "1", d, dp, wp, att_ew)
        lp2 = _prep_gat_params(p, pre + "2", d, dp, wp, att_ew)
        h_pad = _gat_layer(h_pad, edges, lp1, snorm2d, wp)
        h_pad = _gat_layer(h_pad, edges, lp2, snorm2d, wp)
        return h_pad[:, :d]

    h0 = jnp.concatenate([maps_emb, h_emb, gt], axis=-1)
    h = run_stack(h0, "enc", 572, 640, 128, True)
    he = jnp.concatenate([h, gt], axis=-1)
    he = _pallas_matmul(he, p["encl_W"], p["encl_b"], slope=0.01)
    mu = _pallas_matmul(he, p["encmu_W"], p["encmu_b"])
    log_var = _pallas_matmul(he, p["enclv_W"], p["enclv_b"])
    hp0 = jnp.concatenate([maps_emb, h_emb], axis=-1)
    hp = run_stack(hp0, "pri", 512, 512, 128, True)
    hp2 = _pallas_matmul(hp, p["pril_W"], p["pril_b"], slope=0.01)
    mu_p = _pallas_matmul(hp2, p["primu_W"], p["primu_b"])
    log_var_p = _pallas_matmul(hp2, p["prilv_W"], p["prilv_b"])
    eps = jax.random.normal(jax.random.key(42), mu.shape, dtype=jnp.float32)
    z = mu + jnp.exp(0.5 * log_var) * eps
    hd0 = jnp.concatenate([h_emb, z], axis=-1)
    hd = run_stack(hd0, "dec", 384, 384, 128, False)
    recon = _pallas_matmul(jnp.concatenate([hd, z], axis=-1), p["out_W"], p["out_b"])
    return (recon, mu, log_var, mu_p, log_var_p)


# speedup vs baseline: 1.9371x; 1.1302x over previous
import functools

import jax
import jax.numpy as jnp
from jax import lax
from jax.experimental import pallas as pl
from jax.experimental.pallas import tpu as pltpu
from jax.experimental.pallas import tpu_sc as plsc

N = 10000
E = 160000
NROWS_PAD = 10240
CHUNK = 32
PAIR = 2 * CHUNK
E_PAD = 160064
TILE_EDGE_BUF = 5056


def _mm_act_kern(slope, x_ref, w_ref, b_ref, o_ref):
    o = jnp.dot(x_ref[...], w_ref[...], preferred_element_type=jnp.float32)
    o = o + b_ref[...]
    if slope is not None:
        o = jnp.where(o > 0, o, slope * o)
    o_ref[...] = o


def _pallas_matmul(x, w, b, slope=None, block_rows=1000):
    n, k = x.shape
    _, m = w.shape
    return pl.pallas_call(
        functools.partial(_mm_act_kern, slope),
        grid=(n // block_rows,),
        in_specs=[
            pl.BlockSpec((block_rows, k), lambda i: (i, 0)),
            pl.BlockSpec((k, m), lambda i: (0, 0)),
            pl.BlockSpec((m,), lambda i: (0,)),
        ],
        out_specs=pl.BlockSpec((block_rows, m), lambda i: (i, 0)),
        out_shape=jax.ShapeDtypeStruct((n, m), jnp.float32),
    )(x, w, b)


def _gat_mm_kern(npk, x_ref, w_ref, b_ref, as_ref, ad_ref, z_ref, ss_ref, sd_ref):
    po = pl.program_id(1)
    pk = pl.program_id(2)
    part = jnp.dot(x_ref[0], w_ref[0, 0], preferred_element_type=jnp.float32)

    @pl.when(pk == 0)
    def _():
        z_ref[0] = part + b_ref[0]

    @pl.when(pk > 0)
    def _():
        z_ref[0] = z_ref[0] + part

    @pl.when(pk == npk - 1)
    def _():
        zfull = z_ref[0]
        ssp = jnp.dot(zfull, as_ref[0, 0], preferred_element_type=jnp.float32)
        sdp = jnp.dot(zfull, ad_ref[0, 0], preferred_element_type=jnp.float32)

        @pl.when(po == 0)
        def _():
            ss_ref[...] = ssp[:, None]
            sd_ref[...] = sdp[:, None]

        @pl.when(po > 0)
        def _():
            ss_ref[...] = ss_ref[...] + ssp[:, None]
            sd_ref[...] = sd_ref[...] + sdp[:, None]


def _gat_matmul(xp, w4, b2, a_s2, a_d2, wp, block_rows=1000):
    p, n, _ = xp.shape
    grid = (n // block_rows, p, p)
    return pl.pallas_call(
        functools.partial(_gat_mm_kern, p),
        grid=grid,
        in_specs=[
            pl.BlockSpec((1, block_rows, wp), lambda i, po, pk: (pk, i, 0)),
            pl.BlockSpec((1, 1, wp, wp), lambda i, po, pk: (pk, po, 0, 0)),
            pl.BlockSpec((1, 1, wp), lambda i, po, pk: (po, 0, 0)),
            pl.BlockSpec((1, 1, wp), lambda i, po, pk: (po, 0, 0)),
            pl.BlockSpec((1, 1, wp), lambda i, po, pk: (po, 0, 0)),
        ],
        out_specs=[
            pl.BlockSpec((1, block_rows, wp), lambda i, po, pk: (po, i, 0)),
            pl.BlockSpec((block_rows, 1), lambda i, po, pk: (i, 0)),
            pl.BlockSpec((block_rows, 1), lambda i, po, pk: (i, 0)),
        ],
        out_shape=[
            jax.ShapeDtypeStruct((p, n, wp), jnp.float32),
            jax.ShapeDtypeStruct((n, 1), jnp.float32),
            jax.ShapeDtypeStruct((n, 1), jnp.float32),
        ],
    )(xp, w4, b2, a_s2, a_d2)


def _resid_kern(wp, p, dp, h_ref, agg_ref, sn_ref, o_ref):
    a = agg_ref[...]
    s = a[0] + a[1]
    den = s[p][:, 0]
    inv = 1.0 / (den + 1e-9)
    parts = [s[q] for q in range(p)]
    aggc = jnp.concatenate(parts, axis=1)
    o = (h_ref[...] + aggc * inv[:, None]) * sn_ref[...]
    o_ref[...] = jnp.maximum(o, 0.0)


def _gat_resid(h_pad, agg, snorm2d, wp, block_rows=400):
    n, dp = h_pad.shape
    p = dp // wp
    return pl.pallas_call(
        functools.partial(_resid_kern, wp, p, dp),
        grid=(n // block_rows,),
        in_specs=[
            pl.BlockSpec((block_rows, dp), lambda i: (i, 0)),
            pl.BlockSpec((2, p + 1, block_rows, wp), lambda i: (0, 0, i, 0)),
            pl.BlockSpec((block_rows, 1), lambda i: (i, 0)),
        ],
        out_specs=pl.BlockSpec((block_rows, dp), lambda i: (i, 0)),
        out_shape=jax.ShapeDtypeStruct((n, dp), jnp.float32),
    )(h_pad, agg, snorm2d)


@functools.lru_cache(maxsize=None)
def _make_sc_edge_kernel(p):
    wp = 128
    rows_per_tile = NROWS_PAD // 16
    mesh = plsc.VectorSubcoreMesh(core_axis_name="c", subcore_axis_name="s")

    @functools.partial(
        pl.kernel,
        mesh=mesh,
        compiler_params=pltpu.CompilerParams(needs_layout_passes=False),
        out_type=jax.ShapeDtypeStruct((2, p + 1, NROWS_PAD, wp), jnp.float32),
        scratch_types=[
            pltpu.VMEM((TILE_EDGE_BUF,), jnp.int32),
            pltpu.VMEM((TILE_EDGE_BUF,), jnp.int32),
            pltpu.VMEM((TILE_EDGE_BUF,), jnp.float32),
            pltpu.VMEM((TILE_EDGE_BUF,), jnp.float32),
            pltpu.VMEM((16,), jnp.float32),
            pltpu.VMEM((PAIR,), jnp.float32),
            pltpu.VMEM((PAIR,), jnp.float32),
            pltpu.VMEM((CHUNK,), jnp.int32),
            pltpu.VMEM((CHUNK,), jnp.int32),
            pltpu.VMEM((CHUNK, wp), jnp.float32),
            pltpu.VMEM((CHUNK, wp), jnp.float32),
            pltpu.VMEM((CHUNK, wp), jnp.float32),
            pltpu.VMEM((CHUNK, wp), jnp.float32),
            pltpu.VMEM_SHARED((NROWS_PAD, wp), jnp.float32),
            pltpu.SemaphoreType.DMA,
            pltpu.SemaphoreType.DMA,
            pltpu.SemaphoreType.DMA,
            pltpu.SemaphoreType.DMA,
        ],
    )
    def sc_edge(z_hbm, ss_hbm, sd_hbm, src_hbm, dst_hbm, ew_hbm, c_hbm,
                zeros_hbm, agg_hbm,
                src_v, dst_v, ew_v, ex_v, c_v, ssg_v, sdg_v,
                sidx_a, sidx_b,
                rows_a, rows_b, scl_a, scl_b,
                acc_sh, gsem_a, gsem_b, ssem_a, ssem_b):
        cid = lax.axis_index("c")
        sid = lax.axis_index("s")
        wid = sid * 2 + cid

        npairs = jnp.where(wid < 4, 79, 78)
        base_pair = jnp.where(wid < 4, 79 * wid, 316 + 78 * (wid - 4))
        base_e = base_pair * PAIR

        pltpu.sync_copy(src_hbm.at[pl.ds(base_e, TILE_EDGE_BUF)], src_v)
        pltpu.sync_copy(dst_hbm.at[pl.ds(base_e, TILE_EDGE_BUF)], dst_v)
        pltpu.sync_copy(ew_hbm.at[pl.ds(base_e, TILE_EDGE_BUF)], ew_v)
        pltpu.sync_copy(c_hbm, c_v)
        cval = c_v[...]

        def ex_body(g, carry):
            o = g * PAIR
            ga = pltpu.async_copy(ss_hbm.at[src_v.at[pl.ds(o, PAIR)]], ssg_v, gsem_a)
            gb = pltpu.async_copy(sd_hbm.at[dst_v.at[pl.ds(o, PAIR)]], sdg_v, gsem_b)
            ga.wait()
            gb.wait()
            for k in range(PAIR // 16):
                lg = (ssg_v[pl.ds(k * 16, 16)]
                      + sdg_v[pl.ds(k * 16, 16)]
                      + cval * ew_v[pl.ds(o + k * 16, 16)])
                lg = jnp.maximum(lg, 0.2 * lg)
                ex_v[pl.ds(o + k * 16, 16)] = jnp.exp(lg)
            return carry

        lax.fori_loop(0, npairs, ex_body, 0)

        zero16 = jnp.zeros((16,), jnp.int32)

        def scale_chunk(rows_v, scl_v, ebase):
            def r_body(r, carry):
                spl = plsc.load_gather(ex_v, [zero16 + (ebase + r)])
                for k in range(wp // 16):
                    scl_v[r, pl.ds(k * 16, 16)] = rows_v[r, pl.ds(k * 16, 16)] * spl
                return carry
            lax.fori_loop(0, CHUNK, r_body, 0)

        def splat_chunk(scl_v, ebase):
            def r_body(r, carry):
                spl = plsc.load_gather(ex_v, [zero16 + (ebase + r)])
                for k in range(wp // 16):
                    scl_v[r, pl.ds(k * 16, 16)] = spl
                return carry
            lax.fori_loop(0, CHUNK, r_body, 0)

        def stage_idx(dst_buf, src_ref, off):
            for k in range(CHUNK // 16):
                dst_buf[pl.ds(k * 16, 16)] = src_ref[pl.ds(off + k * 16, 16)]

        for q in range(p + 1):
            pltpu.sync_copy(zeros_hbm, acc_sh.at[pl.ds(sid * rows_per_tile, rows_per_tile)])
            plsc.subcore_barrier()

            def wait_scatters(jp):
                @pl.when(jp > 0)
                def _():
                    pltpu.make_async_copy(scl_a, acc_sh.at[sidx_a], ssem_a).wait()
                    pltpu.make_async_copy(scl_b, acc_sh.at[sidx_b], ssem_b).wait()

            if q < p:
                zq = z_hbm.at[q]

                def pair_body(jp, carry):
                    ea = jp * PAIR
                    eb = ea + CHUNK
                    wait_scatters(jp)
                    pltpu.make_async_copy(zq.at[src_v.at[pl.ds(ea, CHUNK)]], rows_a, gsem_a).wait()
                    scale_chunk(rows_a, scl_a, ea)
                    stage_idx(sidx_a, dst_v, ea)
                    pltpu.async_copy(scl_a, acc_sh.at[sidx_a], ssem_a, add=True)

                    @pl.when(jp + 1 < npairs)
                    def _():
                        pltpu.async_copy(
                            zq.at[src_v.at[pl.ds(ea + PAIR, CHUNK)]], rows_a, gsem_a)

                    pltpu.make_async_copy(zq.at[src_v.at[pl.ds(eb, CHUNK)]], rows_b, gsem_b).wait()
                    scale_chunk(rows_b, scl_b, eb)
                    stage_idx(sidx_b, dst_v, eb)
                    pltpu.async_copy(scl_b, acc_sh.at[sidx_b], ssem_b, add=True)

                    @pl.when(jp + 1 < npairs)
                    def _():
                        pltpu.async_copy(
                            zq.at[src_v.at[pl.ds(eb + PAIR, CHUNK)]], rows_b, gsem_b)
                    return carry

                pltpu.async_copy(zq.at[src_v.at[pl.ds(0, CHUNK)]], rows_a, gsem_a)
                pltpu.async_copy(zq.at[src_v.at[pl.ds(CHUNK, CHUNK)]], rows_b, gsem_b)
            else:
                def pair_body(jp, carry):
                    ea = jp * PAIR
                    eb = ea + CHUNK
                    wait_scatters(jp)
                    splat_chunk(scl_a, ea)
                    stage_idx(sidx_a, dst_v, ea)
                    pltpu.async_copy(scl_a, acc_sh.at[sidx_a], ssem_a, add=True)
                    splat_chunk(scl_b, eb)
                    stage_idx(sidx_b, dst_v, eb)
                    pltpu.async_copy(scl_b, acc_sh.at[sidx_b], ssem_b, add=True)
                    return carry

            lax.fori_loop(0, npairs, pair_body, 0)
            pltpu.make_async_copy(scl_a, acc_sh.at[sidx_a], ssem_a).wait()
            pltpu.make_async_copy(scl_b, acc_sh.at[sidx_b], ssem_b).wait()
            plsc.subcore_barrier()
            pltpu.sync_copy(
                acc_sh.at[pl.ds(sid * rows_per_tile, rows_per_tile)],
                agg_hbm.at[cid, q, pl.ds(sid * rows_per_tile, rows_per_tile)],
            )
            if q + 1 < p + 1:
                plsc.subcore_barrier()

    return sc_edge


def _prep_gat_params(pp, pre, d, dp, wp, att_ew):
    p = dp // wp
    wm = pp[pre + "_W"]
    b = pp[pre + "_b"]
    a = pp[pre + "_a"][:, 0]
    w_ext = jnp.pad(wm, ((0, dp - d), (0, dp - d)))
    w4 = w_ext.reshape(p, wp, p, wp).transpose(0, 2, 1, 3)
    b2 = jnp.pad(b, (0, dp - d)).reshape(p, 1, wp)
    a_s2 = jnp.pad(a[:d], (0, dp - d)).reshape(p, 1, wp)
    a_d2 = jnp.pad(a[d:2 * d], (0, dp - d)).reshape(p, 1, wp)
    if att_ew:
        cvec = jnp.full((16,), jnp.sum(a[2 * d:]), jnp.float32)
    else:
        cvec = jnp.zeros((16,), jnp.float32)
    return w4, b2, a_s2, a_d2, cvec


def _gat_layer(h_pad, edges, lp, snorm2d, wp):
    w4, b2, a_s2, a_d2, cvec = lp
    src_p, dst_p, ew_p, zeros_hbm = edges
    n, dp = h_pad.shape
    p = dp // wp
    xp = h_pad.reshape(n, p, wp).transpose(1, 0, 2)
    zpad, ss, sd = _gat_matmul(xp, w4, b2, a_s2, a_d2, wp)
    sc_k = _make_sc_edge_kernel(p)
    agg = sc_k(zpad, ss.reshape(n), sd.reshape(n), src_p, dst_p, ew_p, cvec, zeros_hbm)
    return _gat_resid(h_pad, agg, snorm2d, wp)


def kernel(feats, e_w, snorm_n, gt, maps_emb, params, edge_index):
    p = params
    src = edge_index[0]
    dst = edge_index[1]
    src_p = jnp.pad(src, (0, E_PAD - E))
    dst_p = jnp.pad(dst, (0, E_PAD - E))
    ew_p = jnp.pad(e_w[:, 0], (0, E_PAD - E))
    snorm2d = snorm_n

    h_emb = _pallas_matmul(feats, p["emb_W"], p["emb_b"])

    def run_stack(h0, pre, d, dp, wp, att_ew):
        h_pad = jnp.pad(h0, ((0, 0), (0, dp - d)))
        zeros_hbm = jnp.zeros((NROWS_PAD // 16, wp), jnp.float32)
        edges = (src_p, dst_p, ew_p, zeros_hbm)
        lp1 = _prep_gat_params(p, pre + "1", d, dp, wp, att_ew)
        lp2 = _prep_gat_params(p, pre + "2", d, dp, wp, att_ew)
        h_pad = _gat_layer(h_pad, edges, lp1, snorm2d, wp)
        h_pad = _gat_layer(h_pad, edges, lp2, snorm2d, wp)
        return h_pad[:, :d]

    h0 = jnp.concatenate([maps_emb, h_emb, gt], axis=-1)
    h = run_stack(h0, "enc", 572, 640, 128, True)
    he = jnp.concatenate([h, gt], axis=-1)
    he = _pallas_matmul(he, p["encl_W"], p["encl_b"], slope=0.01)
    mu = _pallas_matmul(he, p["encmu_W"], p["encmu_b"])
    log_var = _pallas_matmul(he, p["enclv_W"], p["enclv_b"])
    hp0 = jnp.concatenate([maps_emb, h_emb], axis=-1)
    hp = run_stack(hp0, "pri", 512, 512, 128, True)
    hp2 = _pallas_matmul(hp, p["pril_W"], p["pril_b"], slope=0.01)
    mu_p = _pallas_matmul(hp2, p["primu_W"], p["primu_b"])
    log_var_p = _pallas_matmul(hp2, p["prilv_W"], p["prilv_b"])
    eps = jax.random.normal(jax.random.key(42), mu.shape, dtype=jnp.float32)
    z = mu + jnp.exp(0.5 * log_var) * eps
    hd0 = jnp.concatenate([h_emb, z], axis=-1)
    hd = run_stack(hd0, "dec", 384, 384, 128, False)
    recon = _pallas_matmul(jnp.concatenate([hd, z], axis=-1), p["out_W"], p["out_b"])
    return (recon, mu, log_var, mu_p, log_var_p)


# speedup vs baseline: 2.0657x; 1.0664x over previous
import functools

import jax
import jax.numpy as jnp
from jax import lax
from jax.experimental import pallas as pl
from jax.experimental.pallas import tpu as pltpu
from jax.experimental.pallas import tpu_sc as plsc

N = 10000
E = 160000
NROWS_PAD = 10240
CHUNK = 32
PAIR = 2 * CHUNK
E_PAD = 160064
TILE_EDGE_BUF = 5056


def _mm_act_kern(slope, x_ref, w_ref, b_ref, o_ref):
    o = jnp.dot(x_ref[...], w_ref[...], preferred_element_type=jnp.float32)
    o = o + b_ref[...]
    if slope is not None:
        o = jnp.where(o > 0, o, slope * o)
    o_ref[...] = o


def _pallas_matmul(x, w, b, slope=None, block_rows=1000):
    n, k = x.shape
    _, m = w.shape
    return pl.pallas_call(
        functools.partial(_mm_act_kern, slope),
        grid=(n // block_rows,),
        in_specs=[
            pl.BlockSpec((block_rows, k), lambda i: (i, 0)),
            pl.BlockSpec((k, m), lambda i: (0, 0)),
            pl.BlockSpec((m,), lambda i: (0,)),
        ],
        out_specs=pl.BlockSpec((block_rows, m), lambda i: (i, 0)),
        out_shape=jax.ShapeDtypeStruct((n, m), jnp.float32),
    )(x, w, b)


def _gat_mm_kern(npk, x_ref, w_ref, b_ref, as_ref, ad_ref, z_ref, ss_ref, sd_ref):
    po = pl.program_id(1)
    pk = pl.program_id(2)
    part = jnp.dot(x_ref[0], w_ref[0, 0], preferred_element_type=jnp.float32)

    @pl.when(pk == 0)
    def _():
        z_ref[0] = part + b_ref[0]

    @pl.when(pk > 0)
    def _():
        z_ref[0] = z_ref[0] + part

    @pl.when(pk == npk - 1)
    def _():
        zfull = z_ref[0]
        ssp = jnp.dot(zfull, as_ref[0, 0], preferred_element_type=jnp.float32)
        sdp = jnp.dot(zfull, ad_ref[0, 0], preferred_element_type=jnp.float32)

        @pl.when(po == 0)
        def _():
            ss_ref[...] = ssp[:, None]
            sd_ref[...] = sdp[:, None]

        @pl.when(po > 0)
        def _():
            ss_ref[...] = ss_ref[...] + ssp[:, None]
            sd_ref[...] = sd_ref[...] + sdp[:, None]


def _gat_matmul(xp, w4, b2, a_s2, a_d2, wp, block_rows=1000):
    p, n, _ = xp.shape
    grid = (n // block_rows, p, p)
    return pl.pallas_call(
        functools.partial(_gat_mm_kern, p),
        grid=grid,
        in_specs=[
            pl.BlockSpec((1, block_rows, wp), lambda i, po, pk: (pk, i, 0)),
            pl.BlockSpec((1, 1, wp, wp), lambda i, po, pk: (pk, po, 0, 0)),
            pl.BlockSpec((1, 1, wp), lambda i, po, pk: (po, 0, 0)),
            pl.BlockSpec((1, 1, wp), lambda i, po, pk: (po, 0, 0)),
            pl.BlockSpec((1, 1, wp), lambda i, po, pk: (po, 0, 0)),
        ],
        out_specs=[
            pl.BlockSpec((1, block_rows, wp), lambda i, po, pk: (po, i, 0)),
            pl.BlockSpec((block_rows, 1), lambda i, po, pk: (i, 0)),
            pl.BlockSpec((block_rows, 1), lambda i, po, pk: (i, 0)),
        ],
        out_shape=[
            jax.ShapeDtypeStruct((p, n, wp), jnp.float32),
            jax.ShapeDtypeStruct((n, 1), jnp.float32),
            jax.ShapeDtypeStruct((n, 1), jnp.float32),
        ],
    )(xp, w4, b2, a_s2, a_d2)


def _resid_kern(wp, p, dp, h_ref, agg_ref, sn_ref, o_ref):
    a = agg_ref[...]
    s = a[0] + a[1]
    den = s[p][:, 0]
    inv = 1.0 / (den + 1e-9)
    parts = [s[q] for q in range(p)]
    aggc = jnp.concatenate(parts, axis=1)
    o = (h_ref[...] + aggc * inv[:, None]) * sn_ref[...]
    o_ref[...] = jnp.maximum(o, 0.0)


def _gat_resid(h_pad, agg, snorm2d, wp, block_rows=400):
    n, dp = h_pad.shape
    p = dp // wp
    return pl.pallas_call(
        functools.partial(_resid_kern, wp, p, dp),
        grid=(n // block_rows,),
        in_specs=[
            pl.BlockSpec((block_rows, dp), lambda i: (i, 0)),
            pl.BlockSpec((2, p + 1, block_rows, wp), lambda i: (0, 0, i, 0)),
            pl.BlockSpec((block_rows, 1), lambda i: (i, 0)),
        ],
        out_specs=pl.BlockSpec((block_rows, dp), lambda i: (i, 0)),
        out_shape=jax.ShapeDtypeStruct((n, dp), jnp.float32),
    )(h_pad, agg, snorm2d)


@functools.lru_cache(maxsize=None)
def _make_sc_edge_kernel(p):
    wp = 128
    rows_per_tile = NROWS_PAD // 16
    mesh = plsc.VectorSubcoreMesh(core_axis_name="c", subcore_axis_name="s")

    @functools.partial(
        pl.kernel,
        mesh=mesh,
        compiler_params=pltpu.CompilerParams(needs_layout_passes=False),
        out_type=jax.ShapeDtypeStruct((2, p + 1, NROWS_PAD, wp), jnp.float32),
        scratch_types=[
            pltpu.VMEM((TILE_EDGE_BUF,), jnp.int32),
            pltpu.VMEM((TILE_EDGE_BUF,), jnp.int32),
            pltpu.VMEM((TILE_EDGE_BUF,), jnp.float32),
            pltpu.VMEM((TILE_EDGE_BUF,), jnp.float32),
            pltpu.VMEM((16,), jnp.float32),
            pltpu.VMEM((PAIR,), jnp.float32),
            pltpu.VMEM((PAIR,), jnp.float32),
            pltpu.VMEM((CHUNK,), jnp.int32),
            pltpu.VMEM((CHUNK,), jnp.int32),
            pltpu.VMEM((CHUNK, wp), jnp.float32),
            pltpu.VMEM((CHUNK, wp), jnp.float32),
            pltpu.VMEM((CHUNK, wp), jnp.float32),
            pltpu.VMEM((CHUNK, wp), jnp.float32),
            pltpu.VMEM_SHARED((NROWS_PAD, wp), jnp.float32),
            pltpu.SemaphoreType.DMA,
            pltpu.SemaphoreType.DMA,
            pltpu.SemaphoreType.DMA,
            pltpu.SemaphoreType.DMA,
        ],
    )
    def sc_edge(z_hbm, ss_hbm, sd_hbm, src_hbm, dst_hbm, ew_hbm, c_hbm,
                zeros_hbm, agg_hbm,
                src_v, dst_v, ew_v, ex_v, c_v, ssg_v, sdg_v,
                sidx_a, sidx_b,
                rows_a, rows_b, scl_a, scl_b,
                acc_sh, gsem_a, gsem_b, ssem_a, ssem_b):
        cid = lax.axis_index("c")
        sid = lax.axis_index("s")
        wid = sid * 2 + cid

        npairs = jnp.where(wid < 4, 79, 78)
        base_pair = jnp.where(wid < 4, 79 * wid, 316 + 78 * (wid - 4))
        base_e = base_pair * PAIR

        pltpu.sync_copy(src_hbm.at[pl.ds(base_e, TILE_EDGE_BUF)], src_v)
        pltpu.sync_copy(dst_hbm.at[pl.ds(base_e, TILE_EDGE_BUF)], dst_v)
        pltpu.sync_copy(ew_hbm.at[pl.ds(base_e, TILE_EDGE_BUF)], ew_v)
        pltpu.sync_copy(c_hbm, c_v)
        cval = c_v[...]

        def ex_body(g, carry):
            o = g * PAIR
            ga = pltpu.async_copy(ss_hbm.at[src_v.at[pl.ds(o, PAIR)]], ssg_v, gsem_a)
            gb = pltpu.async_copy(sd_hbm.at[dst_v.at[pl.ds(o, PAIR)]], sdg_v, gsem_b)
            ga.wait()
            gb.wait()
            for k in range(PAIR // 16):
                lg = (ssg_v[pl.ds(k * 16, 16)]
                      + sdg_v[pl.ds(k * 16, 16)]
                      + cval * ew_v[pl.ds(o + k * 16, 16)])
                lg = jnp.maximum(lg, 0.2 * lg)
                ex_v[pl.ds(o + k * 16, 16)] = jnp.exp(lg)
            return carry

        lax.fori_loop(0, npairs, ex_body, 0)

        zero16 = jnp.zeros((16,), jnp.int32)

        def scale_chunk(rows_v, scl_v, ebase):
            @plsc.parallel_loop(0, CHUNK, 1, unroll=4)
            def r_body(r):
                spl = plsc.load_gather(ex_v, [zero16 + (ebase + r)])
                for k in range(wp // 16):
                    scl_v[r, pl.ds(k * 16, 16)] = rows_v[r, pl.ds(k * 16, 16)] * spl

        def splat_chunk(scl_v, ebase):
            @plsc.parallel_loop(0, CHUNK, 1, unroll=4)
            def r_body(r):
                spl = plsc.load_gather(ex_v, [zero16 + (ebase + r)])
                for k in range(wp // 16):
                    scl_v[r, pl.ds(k * 16, 16)] = spl

        def stage_idx(dst_buf, src_ref, off):
            for k in range(CHUNK // 16):
                dst_buf[pl.ds(k * 16, 16)] = src_ref[pl.ds(off + k * 16, 16)]

        for q in range(p + 1):
            pltpu.sync_copy(zeros_hbm, acc_sh.at[pl.ds(sid * rows_per_tile, rows_per_tile)])
            plsc.subcore_barrier()

            def wait_scatters(jp):
                @pl.when(jp > 0)
                def _():
                    pltpu.make_async_copy(scl_a, acc_sh.at[sidx_a], ssem_a).wait()
                    pltpu.make_async_copy(scl_b, acc_sh.at[sidx_b], ssem_b).wait()

            if q < p:
                zq = z_hbm.at[q]

                def pair_body(jp, carry):
                    ea = jp * PAIR
                    eb = ea + CHUNK
                    wait_scatters(jp)
                    pltpu.make_async_copy(zq.at[src_v.at[pl.ds(ea, CHUNK)]], rows_a, gsem_a).wait()
                    scale_chunk(rows_a, scl_a, ea)
                    stage_idx(sidx_a, dst_v, ea)
                    pltpu.async_copy(scl_a, acc_sh.at[sidx_a], ssem_a, add=True)

                    @pl.when(jp + 1 < npairs)
                    def _():
                        pltpu.async_copy(
                            zq.at[src_v.at[pl.ds(ea + PAIR, CHUNK)]], rows_a, gsem_a)

                    pltpu.make_async_copy(zq.at[src_v.at[pl.ds(eb, CHUNK)]], rows_b, gsem_b).wait()
                    scale_chunk(rows_b, scl_b, eb)
                    stage_idx(sidx_b, dst_v, eb)
                    pltpu.async_copy(scl_b, acc_sh.at[sidx_b], ssem_b, add=True)

                    @pl.when(jp + 1 < npairs)
                    def _():
                        pltpu.async_copy(
                            zq.at[src_v.at[pl.ds(eb + PAIR, CHUNK)]], rows_b, gsem_b)
                    return carry

                pltpu.async_copy(zq.at[src_v.at[pl.ds(0, CHUNK)]], rows_a, gsem_a)
                pltpu.async_copy(zq.at[src_v.at[pl.ds(CHUNK, CHUNK)]], rows_b, gsem_b)
            else:
                def pair_body(jp, carry):
                    ea = jp * PAIR
                    eb = ea + CHUNK
                    wait_scatters(jp)
                    splat_chunk(scl_a, ea)
                    stage_idx(sidx_a, dst_v, ea)
                    pltpu.async_copy(scl_a, acc_sh.at[sidx_a], ssem_a, add=True)
                    splat_chunk(scl_b, eb)
                    stage_idx(sidx_b, dst_v, eb)
                    pltpu.async_copy(scl_b, acc_sh.at[sidx_b], ssem_b, add=True)
                    return carry

            lax.fori_loop(0, npairs, pair_body, 0)
            pltpu.make_async_copy(scl_a, acc_sh.at[sidx_a], ssem_a).wait()
            pltpu.make_async_copy(scl_b, acc_sh.at[sidx_b], ssem_b).wait()
            plsc.subcore_barrier()
            pltpu.sync_copy(
                acc_sh.at[pl.ds(sid * rows_per_tile, rows_per_tile)],
                agg_hbm.at[cid, q, pl.ds(sid * rows_per_tile, rows_per_tile)],
            )
            if q + 1 < p + 1:
                plsc.subcore_barrier()

    return sc_edge


def _prep_gat_params(pp, pre, d, dp, wp, att_ew):
    p = dp // wp
    wm = pp[pre + "_W"]
    b = pp[pre + "_b"]
    a = pp[pre + "_a"][:, 0]
    w_ext = jnp.pad(wm, ((0, dp - d), (0, dp - d)))
    w4 = w_ext.reshape(p, wp, p, wp).transpose(0, 2, 1, 3)
    b2 = jnp.pad(b, (0, dp - d)).reshape(p, 1, wp)
    a_s2 = jnp.pad(a[:d], (0, dp - d)).reshape(p, 1, wp)
    a_d2 = jnp.pad(a[d:2 * d], (0, dp - d)).reshape(p, 1, wp)
    if att_ew:
        cvec = jnp.full((16,), jnp.sum(a[2 * d:]), jnp.float32)
    else:
        cvec = jnp.zeros((16,), jnp.float32)
    return w4, b2, a_s2, a_d2, cvec


def _gat_layer(h_pad, edges, lp, snorm2d, wp):
    w4, b2, a_s2, a_d2, cvec = lp
    src_p, dst_p, ew_p, zeros_hbm = edges
    n, dp = h_pad.shape
    p = dp // wp
    xp = h_pad.reshape(n, p, wp).transpose(1, 0, 2)
    zpad, ss, sd = _gat_matmul(xp, w4, b2, a_s2, a_d2, wp)
    sc_k = _make_sc_edge_kernel(p)
    agg = sc_k(zpad, ss.reshape(n), sd.reshape(n), src_p, dst_p, ew_p, cvec, zeros_hbm)
    return _gat_resid(h_pad, agg, snorm2d, wp)


def kernel(feats, e_w, snorm_n, gt, maps_emb, params, edge_index):
    p = params
    src = edge_index[0]
    dst = edge_index[1]
    src_p = jnp.pad(src, (0, E_PAD - E))
    dst_p = jnp.pad(dst, (0, E_PAD - E))
    ew_p = jnp.pad(e_w[:, 0], (0, E_PAD - E))
    snorm2d = snorm_n

    h_emb = _pallas_matmul(feats, p["emb_W"], p["emb_b"])

    def run_stack(h0, pre, d, dp, wp, att_ew):
        h_pad = jnp.pad(h0, ((0, 0), (0, dp - d)))
        zeros_hbm = jnp.zeros((NROWS_PAD // 16, wp), jnp.float32)
        edges = (src_p, dst_p, ew_p, zeros_hbm)
        lp1 = _prep_gat_params(p, pre + "1", d, dp, wp, att_ew)
        lp2 = _prep_gat_params(p, pre + "2", d, dp, wp, att_ew)
        h_pad = _gat_layer(h_pad, edges, lp1, snorm2d, wp)
        h_pad = _gat_layer(h_pad, edges, lp2, snorm2d, wp)
        return h_pad[:, :d]

    h0 = jnp.concatenate([maps_emb, h_emb, gt], axis=-1)
    h = run_stack(h0, "enc", 572, 640, 128, True)
    he = jnp.concatenate([h, gt], axis=-1)
    he = _pallas_matmul(he, p["encl_W"], p["encl_b"], slope=0.01)
    mu = _pallas_matmul(he, p["encmu_W"], p["encmu_b"])
    log_var = _pallas_matmul(he, p["enclv_W"], p["enclv_b"])
    hp0 = jnp.concatenate([maps_emb, h_emb], axis=-1)
    hp = run_stack(hp0, "pri", 512, 512, 128, True)
    hp2 = _pallas_matmul(hp, p["pril_W"], p["pril_b"], slope=0.01)
    mu_p = _pallas_matmul(hp2, p["primu_W"], p["primu_b"])
    log_var_p = _pallas_matmul(hp2, p["prilv_W"], p["prilv_b"])
    eps = jax.random.normal(jax.random.key(42), mu.shape, dtype=jnp.float32)
    z = mu + jnp.exp(0.5 * log_var) * eps
    hd0 = jnp.concatenate([h_emb, z], axis=-1)
    hd = run_stack(hd0, "dec", 384, 384, 128, False)
    recon = _pallas_matmul(jnp.concatenate([hd, z], axis=-1), p["out_W"], p["out_b"])
    return (recon, mu, log_var, mu_p, log_var_p)


# speedup vs baseline: 2.0757x; 1.0048x over previous
import functools

import jax
import jax.numpy as jnp
from jax import lax
from jax.experimental import pallas as pl
from jax.experimental.pallas import tpu as pltpu
from jax.experimental.pallas import tpu_sc as plsc

N = 10000
E = 160000
NROWS_PAD = 10240
CHUNK = 32
PAIR = 2 * CHUNK
E_PAD = 160064
TILE_EDGE_BUF = 5056


def _mm_act_kern(slope, x_ref, w_ref, b_ref, o_ref):
    o = jnp.dot(x_ref[...], w_ref[...], preferred_element_type=jnp.float32)
    o = o + b_ref[...]
    if slope is not None:
        o = jnp.where(o > 0, o, slope * o)
    o_ref[...] = o


def _pallas_matmul(x, w, b, slope=None, block_rows=1000):
    n, k = x.shape
    _, m = w.shape
    return pl.pallas_call(
        functools.partial(_mm_act_kern, slope),
        grid=(n // block_rows,),
        in_specs=[
            pl.BlockSpec((block_rows, k), lambda i: (i, 0)),
            pl.BlockSpec((k, m), lambda i: (0, 0)),
            pl.BlockSpec((m,), lambda i: (0,)),
        ],
        out_specs=pl.BlockSpec((block_rows, m), lambda i: (i, 0)),
        out_shape=jax.ShapeDtypeStruct((n, m), jnp.float32),
    )(x, w, b)


def _gat_mm_kern(npk, x_ref, w_ref, b_ref, as_ref, ad_ref, z_ref, ss_ref, sd_ref):
    po = pl.program_id(1)
    pk = pl.program_id(2)
    part = jnp.dot(x_ref[0], w_ref[0, 0], preferred_element_type=jnp.float32)

    @pl.when(pk == 0)
    def _():
        z_ref[0] = part + b_ref[0]

    @pl.when(pk > 0)
    def _():
        z_ref[0] = z_ref[0] + part

    @pl.when(pk == npk - 1)
    def _():
        zfull = z_ref[0]
        ssp = jnp.dot(zfull, as_ref[0, 0], preferred_element_type=jnp.float32)
        sdp = jnp.dot(zfull, ad_ref[0, 0], preferred_element_type=jnp.float32)

        @pl.when(po == 0)
        def _():
            ss_ref[...] = ssp[:, None]
            sd_ref[...] = sdp[:, None]

        @pl.when(po > 0)
        def _():
            ss_ref[...] = ss_ref[...] + ssp[:, None]
            sd_ref[...] = sd_ref[...] + sdp[:, None]


def _gat_matmul(xp, w4, b2, a_s2, a_d2, wp, block_rows=1000):
    p, n, _ = xp.shape
    grid = (n // block_rows, p, p)
    return pl.pallas_call(
        functools.partial(_gat_mm_kern, p),
        grid=grid,
        in_specs=[
            pl.BlockSpec((1, block_rows, wp), lambda i, po, pk: (pk, i, 0)),
            pl.BlockSpec((1, 1, wp, wp), lambda i, po, pk: (pk, po, 0, 0)),
            pl.BlockSpec((1, 1, wp), lambda i, po, pk: (po, 0, 0)),
            pl.BlockSpec((1, 1, wp), lambda i, po, pk: (po, 0, 0)),
            pl.BlockSpec((1, 1, wp), lambda i, po, pk: (po, 0, 0)),
        ],
        out_specs=[
            pl.BlockSpec((1, block_rows, wp), lambda i, po, pk: (po, i, 0)),
            pl.BlockSpec((block_rows, 1), lambda i, po, pk: (i, 0)),
            pl.BlockSpec((block_rows, 1), lambda i, po, pk: (i, 0)),
        ],
        out_shape=[
            jax.ShapeDtypeStruct((p, n, wp), jnp.float32),
            jax.ShapeDtypeStruct((n, 1), jnp.float32),
            jax.ShapeDtypeStruct((n, 1), jnp.float32),
        ],
    )(xp, w4, b2, a_s2, a_d2)


def _resid_kern(wp, p, dp, h_ref, agg_ref, sn_ref, o_ref):
    a = agg_ref[...]
    s = a[0] + a[1]
    den = s[p][:, 0]
    inv = 1.0 / (den + 1e-9)
    parts = [s[q] for q in range(p)]
    aggc = jnp.concatenate(parts, axis=1)
    o = (h_ref[...] + aggc * inv[:, None]) * sn_ref[...]
    o_ref[...] = jnp.maximum(o, 0.0)


def _gat_resid(h_pad, agg, snorm2d, wp, block_rows=400):
    n, dp = h_pad.shape
    p = dp // wp
    return pl.pallas_call(
        functools.partial(_resid_kern, wp, p, dp),
        grid=(n // block_rows,),
        in_specs=[
            pl.BlockSpec((block_rows, dp), lambda i: (i, 0)),
            pl.BlockSpec((2, p + 1, block_rows, wp), lambda i: (0, 0, i, 0)),
            pl.BlockSpec((block_rows, 1), lambda i: (i, 0)),
        ],
        out_specs=pl.BlockSpec((block_rows, dp), lambda i: (i, 0)),
        out_shape=jax.ShapeDtypeStruct((n, dp), jnp.float32),
    )(h_pad, agg, snorm2d)


@functools.lru_cache(maxsize=None)
def _make_sc_edge_kernel(p):
    wp = 128
    rows_per_tile = NROWS_PAD // 16
    mesh = plsc.VectorSubcoreMesh(core_axis_name="c", subcore_axis_name="s")

    @functools.partial(
        pl.kernel,
        mesh=mesh,
        compiler_params=pltpu.CompilerParams(needs_layout_passes=False),
        out_type=jax.ShapeDtypeStruct((2, p + 1, NROWS_PAD, wp), jnp.float32),
        scratch_types=[
            pltpu.VMEM((TILE_EDGE_BUF,), jnp.int32),
            pltpu.VMEM((TILE_EDGE_BUF,), jnp.int32),
            pltpu.VMEM((TILE_EDGE_BUF,), jnp.float32),
            pltpu.VMEM((TILE_EDGE_BUF,), jnp.float32),
            pltpu.VMEM((16,), jnp.float32),
            pltpu.VMEM((PAIR,), jnp.float32),
            pltpu.VMEM((PAIR,), jnp.float32),
            pltpu.VMEM((CHUNK,), jnp.int32),
            pltpu.VMEM((CHUNK,), jnp.int32),
            pltpu.VMEM((CHUNK, wp), jnp.float32),
            pltpu.VMEM((CHUNK, wp), jnp.float32),
            pltpu.VMEM((CHUNK, wp), jnp.float32),
            pltpu.VMEM((CHUNK, wp), jnp.float32),
            pltpu.VMEM_SHARED((NROWS_PAD, wp), jnp.float32),
            pltpu.SemaphoreType.DMA,
            pltpu.SemaphoreType.DMA,
            pltpu.SemaphoreType.DMA,
            pltpu.SemaphoreType.DMA,
        ],
    )
    def sc_edge(z_hbm, ss_hbm, sd_hbm, src_hbm, dst_hbm, ew_hbm, c_hbm,
                zeros_hbm, agg_hbm,
                src_v, dst_v, ew_v, ex_v, c_v, ssg_v, sdg_v,
                sidx_a, sidx_b,
                rows_a, rows_b, scl_a, scl_b,
                acc_sh, gsem_a, gsem_b, ssem_a, ssem_b):
        cid = lax.axis_index("c")
        sid = lax.axis_index("s")
        wid = sid * 2 + cid

        npairs = jnp.where(wid < 4, 79, 78)
        base_pair = jnp.where(wid < 4, 79 * wid, 316 + 78 * (wid - 4))
        base_e = base_pair * PAIR

        pltpu.sync_copy(src_hbm.at[pl.ds(base_e, TILE_EDGE_BUF)], src_v)
        pltpu.sync_copy(dst_hbm.at[pl.ds(base_e, TILE_EDGE_BUF)], dst_v)
        pltpu.sync_copy(ew_hbm.at[pl.ds(base_e, TILE_EDGE_BUF)], ew_v)
        pltpu.sync_copy(c_hbm, c_v)
        cval = c_v[...]

        def ex_body(g, carry):
            o = g * PAIR
            ga = pltpu.async_copy(ss_hbm.at[src_v.at[pl.ds(o, PAIR)]], ssg_v, gsem_a)
            gb = pltpu.async_copy(sd_hbm.at[dst_v.at[pl.ds(o, PAIR)]], sdg_v, gsem_b)
            ga.wait()
            gb.wait()
            for k in range(PAIR // 16):
                lg = (ssg_v[pl.ds(k * 16, 16)]
                      + sdg_v[pl.ds(k * 16, 16)]
                      + cval * ew_v[pl.ds(o + k * 16, 16)])
                lg = jnp.maximum(lg, 0.2 * lg)
                ex_v[pl.ds(o + k * 16, 16)] = jnp.exp(lg)
            return carry

        lax.fori_loop(0, npairs, ex_body, 0)

        zero16 = jnp.zeros((16,), jnp.int32)

        def scale_chunk(rows_v, scl_v, ebase):
            @plsc.parallel_loop(0, CHUNK, 1, unroll=8)
            def r_body(r):
                spl = plsc.load_gather(ex_v, [zero16 + (ebase + r)])
                for k in range(wp // 16):
                    scl_v[r, pl.ds(k * 16, 16)] = rows_v[r, pl.ds(k * 16, 16)] * spl

        def splat_chunk(scl_v, ebase):
            @plsc.parallel_loop(0, CHUNK, 1, unroll=8)
            def r_body(r):
                spl = plsc.load_gather(ex_v, [zero16 + (ebase + r)])
                for k in range(wp // 16):
                    scl_v[r, pl.ds(k * 16, 16)] = spl

        def stage_idx(dst_buf, src_ref, off):
            for k in range(CHUNK // 16):
                dst_buf[pl.ds(k * 16, 16)] = src_ref[pl.ds(off + k * 16, 16)]

        for q in range(p + 1):
            pltpu.sync_copy(zeros_hbm, acc_sh.at[pl.ds(sid * rows_per_tile, rows_per_tile)])
            plsc.subcore_barrier()

            def wait_scatters(jp):
                @pl.when(jp > 0)
                def _():
                    pltpu.make_async_copy(scl_a, acc_sh.at[sidx_a], ssem_a).wait()
                    pltpu.make_async_copy(scl_b, acc_sh.at[sidx_b], ssem_b).wait()

            if q < p:
                zq = z_hbm.at[q]

                def pair_body(jp, carry):
                    ea = jp * PAIR
                    eb = ea + CHUNK
                    wait_scatters(jp)
                    pltpu.make_async_copy(zq.at[src_v.at[pl.ds(ea, CHUNK)]], rows_a, gsem_a).wait()
                    scale_chunk(rows_a, scl_a, ea)
                    stage_idx(sidx_a, dst_v, ea)
                    pltpu.async_copy(scl_a, acc_sh.at[sidx_a], ssem_a, add=True)

                    @pl.when(jp + 1 < npairs)
                    def _():
                        pltpu.async_copy(
                            zq.at[src_v.at[pl.ds(ea + PAIR, CHUNK)]], rows_a, gsem_a)

                    pltpu.make_async_copy(zq.at[src_v.at[pl.ds(eb, CHUNK)]], rows_b, gsem_b).wait()
                    scale_chunk(rows_b, scl_b, eb)
                    stage_idx(sidx_b, dst_v, eb)
                    pltpu.async_copy(scl_b, acc_sh.at[sidx_b], ssem_b, add=True)

                    @pl.when(jp + 1 < npairs)
                    def _():
                        pltpu.async_copy(
                            zq.at[src_v.at[pl.ds(eb + PAIR, CHUNK)]], rows_b, gsem_b)
                    return carry

                pltpu.async_copy(zq.at[src_v.at[pl.ds(0, CHUNK)]], rows_a, gsem_a)
                pltpu.async_copy(zq.at[src_v.at[pl.ds(CHUNK, CHUNK)]], rows_b, gsem_b)
            else:
                def pair_body(jp, carry):
                    ea = jp * PAIR
                    eb = ea + CHUNK
                    wait_scatters(jp)
                    splat_chunk(scl_a, ea)
                    stage_idx(sidx_a, dst_v, ea)
                    pltpu.async_copy(scl_a, acc_sh.at[sidx_a], ssem_a, add=True)
                    splat_chunk(scl_b, eb)
                    stage_idx(sidx_b, dst_v, eb)
                    pltpu.async_copy(scl_b, acc_sh.at[sidx_b], ssem_b, add=True)
                    return carry

            lax.fori_loop(0, npairs, pair_body, 0)
            pltpu.make_async_copy(scl_a, acc_sh.at[sidx_a], ssem_a).wait()
            pltpu.make_async_copy(scl_b, acc_sh.at[sidx_b], ssem_b).wait()
            plsc.subcore_barrier()
            pltpu.sync_copy(
                acc_sh.at[pl.ds(sid * rows_per_tile, rows_per_tile)],
                agg_hbm.at[cid, q, pl.ds(sid * rows_per_tile, rows_per_tile)],
            )
            if q + 1 < p + 1:
                plsc.subcore_barrier()

    return sc_edge


def _prep_gat_params(pp, pre, d, dp, wp, att_ew):
    p = dp // wp
    wm = pp[pre + "_W"]
    b = pp[pre + "_b"]
    a = pp[pre + "_a"][:, 0]
    w_ext = jnp.pad(wm, ((0, dp - d), (0, dp - d)))
    w4 = w_ext.reshape(p, wp, p, wp).transpose(0, 2, 1, 3)
    b2 = jnp.pad(b, (0, dp - d)).reshape(p, 1, wp)
    a_s2 = jnp.pad(a[:d], (0, dp - d)).reshape(p, 1, wp)
    a_d2 = jnp.pad(a[d:2 * d], (0, dp - d)).reshape(p, 1, wp)
    if att_ew:
        cvec = jnp.full((16,), jnp.sum(a[2 * d:]), jnp.float32)
    else:
        cvec = jnp.zeros((16,), jnp.float32)
    return w4, b2, a_s2, a_d2, cvec


def _gat_layer(h_pad, edges, lp, snorm2d, wp):
    w4, b2, a_s2, a_d2, cvec = lp
    src_p, dst_p, ew_p, zeros_hbm = edges
    n, dp = h_pad.shape
    p = dp // wp
    xp = h_pad.reshape(n, p, wp).transpose(1, 0, 2)
    zpad, ss, sd = _gat_matmul(xp, w4, b2, a_s2, a_d2, wp)
    sc_k = _make_sc_edge_kernel(p)
    agg = sc_k(zpad, ss.reshape(n), sd.reshape(n), src_p, dst_p, ew_p, cvec, zeros_hbm)
    return _gat_resid(h_pad, agg, snorm2d, wp)


def kernel(feats, e_w, snorm_n, gt, maps_emb, params, edge_index):
    p = params
    src = edge_index[0]
    dst = edge_index[1]
    src_p = jnp.pad(src, (0, E_PAD - E))
    dst_p = jnp.pad(dst, (0, E_PAD - E))
    ew_p = jnp.pad(e_w[:, 0], (0, E_PAD - E))
    snorm2d = snorm_n

    h_emb = _pallas_matmul(feats, p["emb_W"], p["emb_b"])

    def run_stack(h0, pre, d, dp, wp, att_ew):
        h_pad = jnp.pad(h0, ((0, 0), (0, dp - d)))
        zeros_hbm = jnp.zeros((NROWS_PAD // 16, wp), jnp.float32)
        edges = (src_p, dst_p, ew_p, zeros_hbm)
        lp1 = _prep_gat_params(p, pre + "1", d, dp, wp, att_ew)
        lp2 = _prep_gat_params(p, pre + "2", d, dp, wp, att_ew)
        h_pad = _gat_layer(h_pad, edges, lp1, snorm2d, wp)
        h_pad = _gat_layer(h_pad, edges, lp2, snorm2d, wp)
        return h_pad[:, :d]

    h0 = jnp.concatenate([maps_emb, h_emb, gt], axis=-1)
    h = run_stack(h0, "enc", 572, 640, 128, True)
    he = jnp.concatenate([h, gt], axis=-1)
    he = _pallas_matmul(he, p["encl_W"], p["encl_b"], slope=0.01)
    mu = _pallas_matmul(he, p["encmu_W"], p["encmu_b"])
    log_var = _pallas_matmul(he, p["enclv_W"], p["enclv_b"])
    hp0 = jnp.concatenate([maps_emb, h_emb], axis=-1)
    hp = run_stack(hp0, "pri", 512, 512, 128, True)
    hp2 = _pallas_matmul(hp, p["pril_W"], p["pril_b"], slope=0.01)
    mu_p = _pallas_matmul(hp2, p["primu_W"], p["primu_b"])
    log_var_p = _pallas_matmul(hp2, p["prilv_W"], p["prilv_b"])
    eps = jax.random.normal(jax.random.key(42), mu.shape, dtype=jnp.float32)
    z = mu + jnp.exp(0.5 * log_var) * eps
    hd0 = jnp.concatenate([h_emb, z], axis=-1)
    hd = run_stack(hd0, "dec", 384, 384, 128, False)
    recon = _pallas_matmul(jnp.concatenate([hd, z], axis=-1), p["out_W"], p["out_b"])
    return (recon, mu, log_var, mu_p, log_var_p)
